# baseline (speedup 1.0000x reference)
import functools

import jax
import jax.numpy as jnp
from jax import lax
from jax.experimental import pallas as pl
from jax.experimental.pallas import tpu as pltpu
from jax.experimental.pallas import tpu_sc as plsc

_NUM_EMB = 1024
_DIM = 64
_COMMIT = 0.25
_N = 16384
_NW = 32
_BPW = _N // _NW


def _vq_body(flat_ref, embed_ref, x2_ref, e2_ref,
             idx_ref, loss_ref, perp_ref, counts_ref,
             *, m_rows, n_total):
    step = pl.program_id(0)
    last = pl.num_programs(0) - 1

    f = flat_ref[...]
    e = embed_ref[...]
    m = jax.lax.dot_general(f, e, (((1,), (1,)), ((), ())),
                            preferred_element_type=jnp.float32)
    d = x2_ref[...] + e2_ref[...] - 2.0 * m

    dmin = jnp.min(d, axis=1, keepdims=True)
    lanes = jax.lax.broadcasted_iota(jnp.int32, (m_rows, _NUM_EMB), 1)
    idxv = jnp.min(jnp.where(d == dmin, lanes, _NUM_EMB), axis=1,
                   keepdims=True)
    idx_ref[...] = idxv

    onehot = (lanes == idxv).astype(jnp.float32)
    part_counts = jnp.sum(onehot, axis=0, keepdims=True)
    part_loss = jnp.sum(dmin).reshape(1, 1)

    @pl.when(step == 0)
    def _init():
        loss_ref[...] = jnp.zeros((1, 1), jnp.float32)
        counts_ref[...] = jnp.zeros_like(counts_ref)

    loss_ref[...] += part_loss
    counts_ref[...] += part_counts

    @pl.when(step == last)
    def _finish():
        p = counts_ref[...] / jnp.float32(n_total)
        perp_ref[...] = jnp.exp(-jnp.sum(p * jnp.log(p + 1e-10))).reshape(1, 1)
        loss_ref[...] = loss_ref[...] * jnp.float32(_COMMIT / (n_total * _DIM))


def _vq_tc(flat, embed, x2, e2, *, m_rows=512, interpret=False):
    n = flat.shape[0]
    grid = (n // m_rows,)
    out_shapes = (
        jax.ShapeDtypeStruct((n, 1), jnp.int32),
        jax.ShapeDtypeStruct((1, 1), jnp.float32),
        jax.ShapeDtypeStruct((1, 1), jnp.float32),
    )
    return pl.pallas_call(
        functools.partial(_vq_body, m_rows=m_rows, n_total=n),
        grid=grid,
        in_specs=[
            pl.BlockSpec((m_rows, _DIM), lambda i: (i, 0)),
            pl.BlockSpec((_NUM_EMB, _DIM), lambda i: (0, 0)),
            pl.BlockSpec((m_rows, 1), lambda i: (i, 0)),
            pl.BlockSpec((1, _NUM_EMB), lambda i: (0, 0)),
        ],
        out_specs=[
            pl.BlockSpec((m_rows, 1), lambda i: (i, 0)),
            pl.BlockSpec((1, 1), lambda i: (0, 0)),
            pl.BlockSpec((1, 1), lambda i: (0, 0)),
        ],
        out_shape=out_shapes,
        scratch_shapes=[pltpu.VMEM((1, _NUM_EMB), jnp.float32)],
        interpret=interpret,
    )(flat, embed, x2, e2)


@functools.partial(
    pl.kernel,
    out_type=jax.ShapeDtypeStruct((_N, _DIM), jnp.float32),
    mesh=plsc.VectorSubcoreMesh(core_axis_name="c", subcore_axis_name="s"),
    scratch_types=[
        pltpu.VMEM((_BPW,), jnp.int32),
        pltpu.VMEM((_BPW, _DIM), jnp.float32),
        pltpu.SemaphoreType.DMA,
    ],
    compiler_params=pltpu.CompilerParams(use_tc_tiling_on_sc=False),
)
def _sc_gather(idx_hbm, table_hbm, out_hbm, idx_v, rows_v, sem):
    wid = lax.axis_index("s") * 2 + lax.axis_index("c")
    base = wid * _BPW
    pltpu.sync_copy(idx_hbm.at[pl.ds(base, _BPW)], idx_v)
    pltpu.async_copy(table_hbm.at[idx_v], rows_v, sem).wait()
    pltpu.sync_copy(rows_v, out_hbm.at[pl.ds(base, _BPW)])


def kernel(inputs, embed):
    x = jnp.transpose(inputs, (0, 2, 3, 1))
    B, H, W, C = x.shape
    flat = x.reshape(-1, _DIM).astype(jnp.float32)
    embed_f = embed.astype(jnp.float32)
    x2 = jnp.sum(flat ** 2, axis=1, keepdims=True)
    e2 = jnp.sum(embed_f ** 2, axis=1)[None, :]

    idx, loss, perp = _vq_tc(flat, embed_f, x2, e2)
    q = _sc_gather(idx.reshape(-1), embed_f)

    quantized_out = jnp.transpose(q.reshape(B, H, W, C), (0, 3, 1, 2))
    quantized_out = quantized_out.astype(inputs.dtype)
    encoding_indices = idx.reshape(B, H, W)
    return (quantized_out, loss[0, 0], perp[0, 0], encoding_indices)


# speedup vs baseline: 1.6144x; 1.6144x over previous
import functools

import jax
import jax.numpy as jnp
from jax.experimental import pallas as pl
from jax.experimental.pallas import tpu as pltpu

_NUM_EMB = 1024
_DIM = 64
_COMMIT = 0.25
_N = 16384
_HW = 1024
_B = 16


def _vq_body(x_ref, embed_ref, x2_ref, e2_ref,
             q_ref, idx_ref, loss_ref, perp_ref, counts_ref):
    step = pl.program_id(0)
    last = pl.num_programs(0) - 1

    x = x_ref[0]
    e = embed_ref[...]
    m = jax.lax.dot_general(e, x, (((1,), (0,)), ((), ())),
                            preferred_element_type=jnp.float32)
    d = x2_ref[0] + e2_ref[...] - 2.0 * m

    dmin = jnp.min(d, axis=0, keepdims=True)
    subs = jax.lax.broadcasted_iota(jnp.int32, (_NUM_EMB, _HW), 0)
    idxv = jnp.min(jnp.where(d == dmin, subs, _NUM_EMB), axis=0,
                   keepdims=True)
    idx_ref[0] = idxv

    onehot = (subs == idxv).astype(jnp.float32)
    q_ref[0] = jax.lax.dot_general(e, onehot, (((0,), (0,)), ((), ())),
                                   preferred_element_type=jnp.float32)

    part_counts = jnp.sum(onehot, axis=1, keepdims=True)
    part_loss = jnp.sum(dmin).reshape(1, 1)

    @pl.when(step == 0)
    def _init():
        loss_ref[...] = jnp.zeros((1, 1), jnp.float32)
        counts_ref[...] = jnp.zeros_like(counts_ref)

    loss_ref[...] += part_loss
    counts_ref[...] += part_counts

    @pl.when(step == last)
    def _finish():
        p = counts_ref[...] / jnp.float32(_N)
        perp_ref[...] = jnp.exp(-jnp.sum(p * jnp.log(p + 1e-10))).reshape(1, 1)
        loss_ref[...] = loss_ref[...] * jnp.float32(_COMMIT / (_N * _DIM))


def _vq_tc(xb, embed, x2, e2, *, interpret=False):
    out_shapes = (
        jax.ShapeDtypeStruct((_B, _DIM, _HW), jnp.float32),
        jax.ShapeDtypeStruct((_B, 1, _HW), jnp.int32),
        jax.ShapeDtypeStruct((1, 1), jnp.float32),
        jax.ShapeDtypeStruct((1, 1), jnp.float32),
    )
    return pl.pallas_call(
        _vq_body,
        grid=(_B,),
        in_specs=[
            pl.BlockSpec((1, _DIM, _HW), lambda i: (i, 0, 0)),
            pl.BlockSpec((_NUM_EMB, _DIM), lambda i: (0, 0)),
            pl.BlockSpec((1, 1, _HW), lambda i: (i, 0, 0)),
            pl.BlockSpec((_NUM_EMB, 1), lambda i: (0, 0)),
        ],
        out_specs=[
            pl.BlockSpec((1, _DIM, _HW), lambda i: (i, 0, 0)),
            pl.BlockSpec((1, 1, _HW), lambda i: (i, 0, 0)),
            pl.BlockSpec((1, 1), lambda i: (0, 0)),
            pl.BlockSpec((1, 1), lambda i: (0, 0)),
        ],
        out_shape=out_shapes,
        scratch_shapes=[pltpu.VMEM((_NUM_EMB, 1), jnp.float32)],
        interpret=interpret,
    )(xb, embed, x2, e2)


def kernel(inputs, embed):
    B, C, H, W = inputs.shape
    embed_f = embed.astype(jnp.float32)
    xb = inputs.reshape(B, C, H * W).astype(jnp.float32)
    x2 = jnp.sum(jnp.transpose(inputs, (0, 2, 3, 1)).reshape(-1, _DIM) ** 2,
                 axis=1).reshape(B, 1, H * W)
    e2 = jnp.sum(embed_f ** 2, axis=1)[:, None]

    q, idx, loss, perp = _vq_tc(xb, embed_f, x2, e2)

    quantized_out = q.reshape(B, C, H, W).astype(inputs.dtype)
    encoding_indices = idx.reshape(B, H, W)
    return (quantized_out, loss[0, 0], perp[0, 0], encoding_indices)


# speedup vs baseline: 1.6266x; 1.0075x over previous
import functools

import jax
import jax.numpy as jnp
from jax.experimental import pallas as pl
from jax.experimental.pallas import tpu as pltpu

_NUM_EMB = 1024
_DIM = 64
_COMMIT = 0.25
_N = 16384
_HW = 1024
_B = 16


def _vq_body(x_ref, embed_ref, x2_ref, e2_ref,
             q_ref, idx_ref, loss_ref, perp_ref, counts_ref, *, bb):
    step = pl.program_id(0)
    last = pl.num_programs(0) - 1

    e = embed_ref[...]
    subs = jax.lax.broadcasted_iota(jnp.int32, (_NUM_EMB, _HW), 0)
    part_loss = jnp.zeros((1, 1), jnp.float32)
    part_counts = jnp.zeros((_NUM_EMB, 1), jnp.float32)
    for k in range(bb):
        x = x_ref[k]
        m = jax.lax.dot_general(e, x, (((1,), (0,)), ((), ())),
                                preferred_element_type=jnp.float32)
        d = x2_ref[k] + e2_ref[...] - 2.0 * m

        dmin = jnp.min(d, axis=0, keepdims=True)
        idxv = jnp.min(jnp.where(d == dmin, subs, _NUM_EMB), axis=0,
                       keepdims=True)
        idx_ref[k] = idxv

        onehot = (subs == idxv).astype(jnp.float32)
        q_ref[k] = jax.lax.dot_general(e, onehot, (((0,), (0,)), ((), ())),
                                       preferred_element_type=jnp.float32)

        part_counts += jnp.sum(onehot, axis=1, keepdims=True)
        part_loss += jnp.sum(dmin).reshape(1, 1)

    @pl.when(step == 0)
    def _init():
        loss_ref[...] = jnp.zeros((1, 1), jnp.float32)
        counts_ref[...] = jnp.zeros_like(counts_ref)

    loss_ref[...] += part_loss
    counts_ref[...] += part_counts

    @pl.when(step == last)
    def _finish():
        p = counts_ref[...] / jnp.float32(_N)
        perp_ref[...] = jnp.exp(-jnp.sum(p * jnp.log(p + 1e-10))).reshape(1, 1)
        loss_ref[...] = loss_ref[...] * jnp.float32(_COMMIT / (_N * _DIM))


def _vq_tc(xb, embed, x2, e2, *, bb=2, interpret=False):
    out_shapes = (
        jax.ShapeDtypeStruct((_B, _DIM, _HW), jnp.float32),
        jax.ShapeDtypeStruct((_B, 1, _HW), jnp.int32),
        jax.ShapeDtypeStruct((1, 1), jnp.float32),
        jax.ShapeDtypeStruct((1, 1), jnp.float32),
    )
    return pl.pallas_call(
        functools.partial(_vq_body, bb=bb),
        grid=(_B // bb,),
        in_specs=[
            pl.BlockSpec((bb, _DIM, _HW), lambda i: (i, 0, 0)),
            pl.BlockSpec((_NUM_EMB, _DIM), lambda i: (0, 0)),
            pl.BlockSpec((bb, 1, _HW), lambda i: (i, 0, 0)),
            pl.BlockSpec((_NUM_EMB, 1), lambda i: (0, 0)),
        ],
        out_specs=[
            pl.BlockSpec((bb, _DIM, _HW), lambda i: (i, 0, 0)),
            pl.BlockSpec((bb, 1, _HW), lambda i: (i, 0, 0)),
            pl.BlockSpec((1, 1), lambda i: (0, 0)),
            pl.BlockSpec((1, 1), lambda i: (0, 0)),
        ],
        out_shape=out_shapes,
        scratch_shapes=[pltpu.VMEM((_NUM_EMB, 1), jnp.float32)],
        interpret=interpret,
    )(xb, embed, x2, e2)


def kernel(inputs, embed):
    B, C, H, W = inputs.shape
    embed_f = embed.astype(jnp.float32)
    xb = inputs.reshape(B, C, H * W).astype(jnp.float32)
    x2 = jnp.sum(jnp.transpose(inputs, (0, 2, 3, 1)).reshape(-1, _DIM) ** 2,
                 axis=1).reshape(B, 1, H * W)
    e2 = jnp.sum(embed_f ** 2, axis=1)[:, None]

    q, idx, loss, perp = _vq_tc(xb, embed_f, x2, e2)

    quantized_out = q.reshape(B, C, H, W).astype(inputs.dtype)
    encoding_indices = idx.reshape(B, H, W)
    return (quantized_out, loss[0, 0], perp[0, 0], encoding_indices)


# speedup vs baseline: 1.7275x; 1.0621x over previous
import functools

import jax
import jax.numpy as jnp
from jax.experimental import pallas as pl
from jax.experimental.pallas import tpu as pltpu

_NUM_EMB = 1024
_DIM = 64
_COMMIT = 0.25
_N = 16384
_HW = 1024
_B = 16


def _vq_body(x_ref, embed_ref, x2_ref, e2_ref,
             q_ref, idx_ref, loss_ref, perp_ref, counts_ref, *, bb):
    step = pl.program_id(0)
    last = pl.num_programs(0) - 1

    e = embed_ref[...]
    em = -2.0 * e
    subs = jax.lax.broadcasted_iota(jnp.int32, (_NUM_EMB, _HW), 0
                                    ).astype(jnp.float32)
    part_loss = jnp.zeros((1, 1), jnp.float32)
    part_counts = jnp.zeros((_NUM_EMB, 1), jnp.float32)
    for k in range(bb):
        x = x_ref[k]
        m2 = jax.lax.dot_general(em, x, (((1,), (0,)), ((), ())),
                                 preferred_element_type=jnp.float32)
        d = (x2_ref[k] + e2_ref[...]) + m2

        dmin = jnp.min(d, axis=0, keepdims=True)
        idxf = jnp.min(jnp.where(d == dmin, subs, jnp.float32(_NUM_EMB)),
                       axis=0, keepdims=True)
        idxv = idxf.astype(jnp.int32)
        idx_ref[k] = idxv

        onehot = (subs == idxf).astype(jnp.float32)
        q_ref[k] = jax.lax.dot_general(e, onehot, (((0,), (0,)), ((), ())),
                                       preferred_element_type=jnp.float32)

        part_counts += jnp.sum(onehot, axis=1, keepdims=True)
        part_loss += jnp.sum(dmin).reshape(1, 1)

    @pl.when(step == 0)
    def _init():
        loss_ref[...] = jnp.zeros((1, 1), jnp.float32)
        counts_ref[...] = jnp.zeros_like(counts_ref)

    loss_ref[...] += part_loss
    counts_ref[...] += part_counts

    @pl.when(step == last)
    def _finish():
        p = counts_ref[...] / jnp.float32(_N)
        perp_ref[...] = jnp.exp(-jnp.sum(p * jnp.log(p + 1e-10))).reshape(1, 1)
        loss_ref[...] = loss_ref[...] * jnp.float32(_COMMIT / (_N * _DIM))


def _vq_tc(xb, embed, x2, e2, *, bb=2, interpret=False):
    out_shapes = (
        jax.ShapeDtypeStruct((_B, _DIM, _HW), jnp.float32),
        jax.ShapeDtypeStruct((_B, 1, _HW), jnp.int32),
        jax.ShapeDtypeStruct((1, 1), jnp.float32),
        jax.ShapeDtypeStruct((1, 1), jnp.float32),
    )
    return pl.pallas_call(
        functools.partial(_vq_body, bb=bb),
        grid=(_B // bb,),
        in_specs=[
            pl.BlockSpec((bb, _DIM, _HW), lambda i: (i, 0, 0)),
            pl.BlockSpec((_NUM_EMB, _DIM), lambda i: (0, 0)),
            pl.BlockSpec((bb, 1, _HW), lambda i: (i, 0, 0)),
            pl.BlockSpec((_NUM_EMB, 1), lambda i: (0, 0)),
        ],
        out_specs=[
            pl.BlockSpec((bb, _DIM, _HW), lambda i: (i, 0, 0)),
            pl.BlockSpec((bb, 1, _HW), lambda i: (i, 0, 0)),
            pl.BlockSpec((1, 1), lambda i: (0, 0)),
            pl.BlockSpec((1, 1), lambda i: (0, 0)),
        ],
        out_shape=out_shapes,
        scratch_shapes=[pltpu.VMEM((_NUM_EMB, 1), jnp.float32)],
        interpret=interpret,
    )(xb, embed, x2, e2)


def kernel(inputs, embed):
    B, C, H, W = inputs.shape
    embed_f = embed.astype(jnp.float32)
    xb = inputs.reshape(B, C, H * W).astype(jnp.float32)
    x2 = jnp.sum(jnp.transpose(inputs, (0, 2, 3, 1)).reshape(-1, _DIM) ** 2,
                 axis=1).reshape(B, 1, H * W)
    e2 = jnp.sum(embed_f ** 2, axis=1)[:, None]

    q, idx, loss, perp = _vq_tc(xb, embed_f, x2, e2)

    quantized_out = q.reshape(B, C, H, W).astype(inputs.dtype)
    encoding_indices = idx.reshape(B, H, W)
    return (quantized_out, loss[0, 0], perp[0, 0], encoding_indices)
